# baseline (device time: 15624 ns/iter reference)
import jax
import jax.numpy as jnp
from jax import lax
from jax.experimental import pallas as pl
from jax.experimental.pallas import tpu as pltpu

N_DEV = 16
N_CHUNKS = 8


def kernel(x):
    m_per, n = x.shape
    chunk_m = m_per // N_CHUNKS

    def body(x_hbm, out_ref, bufs, comm_ref, copy_sems, send_sems, recv_sems):
        my_pos = lax.axis_index("i")

        barrier_sem = pltpu.get_barrier_semaphore()
        for d in range(1, N_DEV):
            peer = lax.rem(my_pos + d, N_DEV)
            pl.semaphore_signal(
                barrier_sem,
                inc=1,
                device_id=(peer,),
                device_id_type=pl.DeviceIdType.MESH,
            )

        copies = []
        for c in range(N_CHUNKS):
            cp = pltpu.make_async_copy(
                x_hbm.at[pl.ds(c * chunk_m, chunk_m), :],
                bufs.at[c],
                copy_sems.at[c],
            )
            cp.start()
            copies.append(cp)

        copies[0].wait()
        acc = jnp.sum(bufs[0], axis=0, keepdims=True)
        for c in range(1, N_CHUNKS):
            copies[c].wait()
            acc = acc + jnp.sum(bufs[c], axis=0, keepdims=True)
        comm_ref[pl.ds(my_pos, 1), :] = acc

        pl.semaphore_wait(barrier_sem, N_DEV - 1)

        for d in range(1, N_DEV):
            peer = lax.rem(my_pos + d, N_DEV)
            rdma = pltpu.make_async_remote_copy(
                src_ref=comm_ref.at[pl.ds(my_pos, 1), :],
                dst_ref=comm_ref.at[pl.ds(my_pos, 1), :],
                send_sem=send_sems.at[d],
                recv_sem=recv_sems.at[d],
                device_id=(peer,),
                device_id_type=pl.DeviceIdType.MESH,
            )
            rdma.start()

        for d in range(1, N_DEV):
            src_pos = lax.rem(my_pos - d + N_DEV, N_DEV)
            recv = pltpu.make_async_remote_copy(
                src_ref=comm_ref.at[pl.ds(my_pos, 1), :],
                dst_ref=comm_ref.at[pl.ds(src_pos, 1), :],
                send_sem=send_sems.at[d],
                recv_sem=recv_sems.at[d],
                device_id=(src_pos,),
                device_id_type=pl.DeviceIdType.MESH,
            )
            recv.wait_recv()

        out_ref[...] = jnp.sum(comm_ref[...], axis=0, keepdims=True)

        for d in range(1, N_DEV):
            send = pltpu.make_async_remote_copy(
                src_ref=comm_ref.at[pl.ds(my_pos, 1), :],
                dst_ref=comm_ref.at[pl.ds(my_pos, 1), :],
                send_sem=send_sems.at[d],
                recv_sem=recv_sems.at[d],
                device_id=(0,),
                device_id_type=pl.DeviceIdType.MESH,
            )
            send.wait_send()

    return pl.pallas_call(
        body,
        out_shape=jax.ShapeDtypeStruct((1, n), jnp.float32),
        in_specs=[pl.BlockSpec(memory_space=pltpu.MemorySpace.HBM)],
        out_specs=pl.BlockSpec(memory_space=pltpu.VMEM),
        scratch_shapes=[
            pltpu.VMEM((N_CHUNKS, chunk_m, n), jnp.float32),
            pltpu.VMEM((N_DEV, n), jnp.float32),
            pltpu.SemaphoreType.DMA((N_CHUNKS,)),
            pltpu.SemaphoreType.DMA((N_DEV,)),
            pltpu.SemaphoreType.DMA((N_DEV,)),
        ],
        compiler_params=pltpu.CompilerParams(
            collective_id=0,
            vmem_limit_bytes=64 * 1024 * 1024,
        ),
    )(x)
